# baseline (speedup 1.0000x reference)
import functools

import jax
import jax.numpy as jnp
from jax import lax
from jax.experimental import pallas as pl
from jax.experimental.pallas import tpu as pltpu
from jax.experimental.pallas import tpu_sc as plsc

N = 4096
S = 7781
IMG = 3 * 64 * 64
KEPT = N - N // 8
NC, NS, L = 2, 16, 16
NW = NC * NS
OUT_PER_W = KEPT // NW
CHUNK = 8
CHUNKS = OUT_PER_W // CHUNK

_MASK_BR = 128


def _mask_body(spec_ref, mask_ref):
    x = spec_ref[...]
    nz = jnp.any(x != 0.0, axis=1)
    mask_ref[0, 0, :] = nz.astype(jnp.int32)


def _compute_mask(spectrum):
    nb = N // _MASK_BR
    mask3 = pl.pallas_call(
        _mask_body,
        grid=(nb,),
        in_specs=[pl.BlockSpec((_MASK_BR, S), lambda i: (i, 0))],
        out_specs=pl.BlockSpec((1, 1, _MASK_BR), lambda i: (i, 0, 0)),
        out_shape=jax.ShapeDtypeStruct((nb, 1, _MASK_BR), jnp.int32),
    )(spectrum)
    return mask3.reshape(N)


_mesh = plsc.VectorSubcoreMesh(core_axis_name="c", subcore_axis_name="s")


@functools.partial(
    pl.kernel,
    out_type=(
        jax.ShapeDtypeStruct((KEPT,), jnp.int32),
        jax.ShapeDtypeStruct((KEPT, IMG), jnp.float32),
        jax.ShapeDtypeStruct((KEPT,), jnp.int32),
        jax.ShapeDtypeStruct((KEPT,), jnp.float32),
    ),
    mesh=_mesh,
    scratch_types=[
        pltpu.VMEM((N,), jnp.int32),
        pltpu.VMEM((KEPT,), jnp.int32),
        pltpu.VMEM((CHUNK, IMG), jnp.float32),
        pltpu.VMEM((N,), jnp.int32),
        pltpu.VMEM((N,), jnp.float32),
        pltpu.VMEM((OUT_PER_W,), jnp.int32),
        pltpu.VMEM((OUT_PER_W,), jnp.float32),
        pltpu.SemaphoreType.DMA,
    ],
    compiler_params=pltpu.CompilerParams(needs_layout_passes=False),
)
def _sc_compact(mask_hbm, img_hbm, tid_hbm, rs_hbm,
                idx_out, img_out, tid_out, rs_out,
                mask_v, idx_v, img_bufs, tid_v, rs_v,
                tid_ov, rs_ov, sem_g):
    wid = lax.axis_index("s") * NC + lax.axis_index("c")
    obase = wid * OUT_PER_W

    pltpu.sync_copy(mask_hbm, mask_v)
    pltpu.sync_copy(tid_hbm, tid_v)
    pltpu.sync_copy(rs_hbm, rs_v)

    zeros16 = jnp.zeros((L,), jnp.int32)

    def _zero(i, carry):
        idx_v[pl.ds(i * L, L)] = zeros16
        return carry

    lax.fori_loop(0, KEPT // L, _zero, 0)

    iota16 = lax.iota(jnp.int32, L)

    def _scan(c, carry):
        m = mask_v[pl.ds(c * L, L)]
        s = plsc.cumsum(m)
        pos = carry + s - m
        vals = c * L + iota16
        plsc.store_scatter(idx_v, [pos], vals, mask=m != 0)
        return carry + jnp.sum(m)

    lax.fori_loop(0, N // L, _scan, jnp.int32(0))

    pltpu.sync_copy(idx_v.at[pl.ds(obase, OUT_PER_W)],
                    idx_out.at[pl.ds(obase, OUT_PER_W)])

    for c in range(CHUNKS):
        pltpu.async_copy(
            img_hbm.at[idx_v.at[pl.ds(obase + c * CHUNK, CHUNK)]],
            img_bufs, sem_g).wait()
        pltpu.sync_copy(img_bufs,
                        img_out.at[pl.ds(obase + c * CHUNK, CHUNK)])

    for v in range(OUT_PER_W // L):
        ids = idx_v[pl.ds(obase + v * L, L)]
        tid_ov[pl.ds(v * L, L)] = plsc.load_gather(tid_v, [ids])
        rs_ov[pl.ds(v * L, L)] = plsc.load_gather(rs_v, [ids])
    pltpu.sync_copy(tid_ov, tid_out.at[pl.ds(obase, OUT_PER_W)])
    pltpu.sync_copy(rs_ov, rs_out.at[pl.ds(obase, OUT_PER_W)])


def _spec_gather_body(idx_ref, spec_ref, out_ref):
    out_ref[...] = spec_ref[...]


def _gather_spectrum(idx, spectrum):
    grid_spec = pltpu.PrefetchScalarGridSpec(
        num_scalar_prefetch=1,
        grid=(KEPT,),
        in_specs=[pl.BlockSpec((1, 1, S),
                               lambda i, idx_ref: (idx_ref[i], 0, 0))],
        out_specs=pl.BlockSpec((1, 1, S), lambda i, idx_ref: (i, 0, 0)),
    )
    out = pl.pallas_call(
        _spec_gather_body,
        grid_spec=grid_spec,
        out_shape=jax.ShapeDtypeStruct((KEPT, 1, S), jnp.float32),
    )(idx, spectrum.reshape(N, 1, S))
    return out.reshape(KEPT, S)


def kernel(spectrum, image, targetid, redshift):
    mask = _compute_mask(spectrum)
    img2 = image.reshape(N, IMG)
    idx, img_o, tid_o, rs_o = _sc_compact(mask, img2, targetid, redshift)
    spec_o = _gather_spectrum(idx, spectrum)
    return spec_o, img_o.reshape(KEPT, 3, 64, 64), tid_o, rs_o
